# Initial kernel scaffold; baseline (speedup 1.0000x reference)
import jax
import jax.numpy as jnp
from jax.experimental import pallas as pl


def kernel(x, Wq, Wk, Wv, Wp):
    raise NotImplementedError("write your pallas kernel here")


# speedup vs baseline: 4.8613x; 4.8613x over previous
import functools
import math

import jax
import jax.numpy as jnp
from jax import lax
from jax.experimental import pallas as pl
from jax.experimental.pallas import tpu as pltpu
from jax.experimental.pallas import tpu_sc as plsc

S = 4096
D = 1024
M = 1024
TOPK = 32

NC = 2
NS = 16
L = 16
NW = NC * NS
ROWS_PER_W = S // NW
GROUPS = ROWS_PER_W // L

SBLK = 512
NSB = S // SBLK

_UHI = 0x80000000


def _pool_slots(x):
    import numpy as np
    b, s, d = x.shape
    idx = np.arange(M)
    starts = (idx * s) // M
    ends = -((-(idx + 1) * s) // M)
    csum = jnp.concatenate(
        [jnp.zeros((b, 1, d), dtype=x.dtype), jnp.cumsum(x, axis=1)], axis=1)
    counts = jnp.asarray(ends - starts, dtype=x.dtype)
    return (csum[:, ends, :] - csum[:, starts, :]) / counts[None, :, None]


def _kv_body(mem_ref, wk_ref, wv_ref, wp_ref, k_ref, vp_ref):
    mem = mem_ref[...]
    k_ref[...] = lax.dot_general(mem, wk_ref[...], (((1,), (1,)), ((), ())),
                                 preferred_element_type=jnp.float32)
    v = lax.dot_general(mem, wv_ref[...], (((1,), (1,)), ((), ())),
                        preferred_element_type=jnp.float32)
    vp_ref[...] = lax.dot_general(v, wp_ref[...], (((1,), (1,)), ((), ())),
                                  preferred_element_type=jnp.float32)


def _kv_call(mem2d, Wk, Wv, Wp):
    mblk = M // NSB
    return pl.pallas_call(
        _kv_body,
        grid=(NSB,),
        in_specs=[
            pl.BlockSpec((mblk, D), lambda i: (i, 0)),
            pl.BlockSpec((D, D), lambda i: (0, 0)),
            pl.BlockSpec((D, D), lambda i: (0, 0)),
            pl.BlockSpec((D, D), lambda i: (0, 0)),
        ],
        out_specs=[
            pl.BlockSpec((mblk, D), lambda i: (i, 0)),
            pl.BlockSpec((mblk, D), lambda i: (i, 0)),
        ],
        out_shape=[
            jax.ShapeDtypeStruct((M, D), jnp.float32),
            jax.ShapeDtypeStruct((M, D), jnp.float32),
        ],
    )(mem2d, Wk, Wv, Wp)


def _scores_body(x_ref, wq_ref, k_ref, st_ref):
    q = lax.dot_general(x_ref[...], wq_ref[...], (((1,), (1,)), ((), ())),
                        preferred_element_type=jnp.float32)
    st = lax.dot_general(k_ref[...], q, (((1,), (1,)), ((), ())),
                         preferred_element_type=jnp.float32)
    st_ref[...] = st * (1.0 / math.sqrt(D))


def _scores_call(x2d, Wq, kmat):
    return pl.pallas_call(
        _scores_body,
        grid=(NSB,),
        in_specs=[
            pl.BlockSpec((SBLK, D), lambda i: (i, 0)),
            pl.BlockSpec((D, D), lambda i: (0, 0)),
            pl.BlockSpec((M, D), lambda i: (0, 0)),
        ],
        out_specs=pl.BlockSpec((M, SBLK), lambda i: (0, i)),
        out_shape=jax.ShapeDtypeStruct((M, S), jnp.float32),
    )(x2d, Wq, kmat)


def _sc_topk_body(st_hbm, tau_hbm, mx_hbm, tile_f, tile_k, hist, tau_buf,
                  mx_buf):
    wid = lax.axis_index("s") * NC + lax.axis_index("c")
    lane = lax.iota(jnp.int32, L)
    ones = jnp.ones((L,), jnp.int32)
    zeros_i = jnp.zeros((L,), jnp.int32)

    def zbody(b, _):
        hist[pl.ds(b * L, L)] = zeros_i
        return 0
    lax.fori_loop(0, 256, zbody, 0)

    def group_body(j, _):
        col0 = (wid * GROUPS + j) * L
        pltpu.sync_copy(st_hbm.at[:, pl.ds(col0, L)], tile_f)

        def kbody(m, fmx):
            v = tile_f[m, :]
            b = plsc.bitcast(v, jnp.uint32)
            key = jnp.where(b >= jnp.uint32(_UHI), ~b, b | jnp.uint32(_UHI))
            tile_k[m, :] = plsc.bitcast(key, jnp.int32)
            return jnp.maximum(fmx, v)
        fmax = lax.fori_loop(0, M, kbody,
                             jnp.full((L,), -jnp.inf, jnp.float32))

        prefix = jnp.zeros((L,), jnp.uint32)
        rank = jnp.full((L,), TOPK, jnp.int32)
        for p in range(4):
            shift = 24 - 8 * p
            himask = jnp.uint32((0xFFFFFFFF << (shift + 8)) & 0xFFFFFFFF
                                if p > 0 else 0)
            shift_u = jnp.uint32(shift)

            def hbody(m, _, himask=himask, shift_u=shift_u, prefix=prefix):
                key = plsc.bitcast(tile_k[m, :], jnp.uint32)
                cand = (key & himask) == prefix
                byte = (key >> shift_u) & jnp.uint32(0xFF)
                addr = plsc.bitcast(byte, jnp.int32) * L + lane
                plsc.addupdate_scatter(hist, [addr], ones, mask=cand)
                return 0
            lax.fori_loop(0, M, hbody, 0)

            def sbody(i, sc, rank=rank):
                cum, found, bsel, rrem = sc
                b = 255 - i
                cnt = hist[pl.ds(b * L, L)]
                hist[pl.ds(b * L, L)] = zeros_i
                newcum = cum + cnt
                nf = jnp.logical_and(found == 0, newcum >= rank)
                bsel = jnp.where(nf, b, bsel)
                rrem = jnp.where(nf, rank - cum, rrem)
                found = jnp.where(nf, 1, found)
                return newcum, found, bsel, rrem
            _, _, bsel, rrem = lax.fori_loop(
                0, 256, sbody, (zeros_i, zeros_i, zeros_i, zeros_i))

            prefix = prefix | (plsc.bitcast(bsel, jnp.uint32) << shift_u)
            rank = rrem

        bits = jnp.where(prefix >= jnp.uint32(_UHI), prefix ^ jnp.uint32(_UHI), ~prefix)
        tau = plsc.bitcast(bits, jnp.float32)
        tau_buf[pl.ds(j * L, L)] = tau
        mx_buf[pl.ds(j * L, L)] = fmax
        return 0

    lax.fori_loop(0, GROUPS, group_body, 0)

    pltpu.sync_copy(tau_buf, tau_hbm.at[pl.ds(wid * ROWS_PER_W, ROWS_PER_W)])
    pltpu.sync_copy(mx_buf, mx_hbm.at[pl.ds(wid * ROWS_PER_W, ROWS_PER_W)])


def _sc_topk_call(scoresT):
    mesh = plsc.VectorSubcoreMesh(core_axis_name="c", subcore_axis_name="s")
    fn = functools.partial(
        pl.kernel,
        out_type=(jax.ShapeDtypeStruct((S,), jnp.float32),
                  jax.ShapeDtypeStruct((S,), jnp.float32)),
        mesh=mesh,
        scratch_types=[
            pltpu.VMEM((M, L), jnp.float32),
            pltpu.VMEM((M, L), jnp.int32),
            pltpu.VMEM((256 * L,), jnp.int32),
            pltpu.VMEM((ROWS_PER_W,), jnp.float32),
            pltpu.VMEM((ROWS_PER_W,), jnp.float32),
        ],
        compiler_params=pltpu.CompilerParams(use_tc_tiling_on_sc=False,
                                             needs_layout_passes=False),
    )(_sc_topk_body)
    return fn(scoresT)


def _combine_body(st_ref, tau_ref, mx_ref, vp_ref, out_ref):
    st = st_ref[...]
    tau = tau_ref[...]
    mx = mx_ref[...]
    w = jnp.where(st >= tau, jnp.exp(st - mx), 0.0)
    z = jnp.sum(w, axis=0, keepdims=True)
    wn = w / z
    out_ref[...] = lax.dot_general(wn, vp_ref[...], (((0,), (0,)), ((), ())),
                                   preferred_element_type=jnp.float32)


def _combine_call(scoresT, tau, mx, vp):
    return pl.pallas_call(
        _combine_body,
        grid=(NSB,),
        in_specs=[
            pl.BlockSpec((M, SBLK), lambda i: (0, i)),
            pl.BlockSpec((1, SBLK), lambda i: (0, i)),
            pl.BlockSpec((1, SBLK), lambda i: (0, i)),
            pl.BlockSpec((M, D), lambda i: (0, 0)),
        ],
        out_specs=pl.BlockSpec((SBLK, D), lambda i: (i, 0)),
        out_shape=jax.ShapeDtypeStruct((S, D), jnp.float32),
    )(scoresT, tau, mx, vp)


def kernel(x, Wq, Wk, Wv, Wp):
    b, s, d = x.shape
    assert (b, s, d) == (1, S, D)
    x2d = x.reshape(S, D)
    mem2d = _pool_slots(x).reshape(M, D)
    kmat, vp = _kv_call(mem2d, Wk, Wv, Wp)
    scoresT = _scores_call(x2d, Wq, kmat)
    tau, mx = _sc_topk_call(scoresT)
    out = _combine_call(scoresT, tau.reshape(1, S), mx.reshape(1, S), vp)
    return out.reshape(1, S, D)


# speedup vs baseline: 6.1167x; 1.2583x over previous
import functools
import math

import numpy as np

import jax
import jax.numpy as jnp
from jax import lax
from jax.experimental import pallas as pl
from jax.experimental.pallas import tpu as pltpu
from jax.experimental.pallas import tpu_sc as plsc

S = 4096
D = 1024
M = 1024
TOPK = 32

NC = 2
NS = 16
L = 16
NW = NC * NS
ROWS_PER_W = S // NW
GROUPS = ROWS_PER_W // L

SBLK = 512
NSB = S // SBLK

_UHI = 0x80000000


def _pool_slots(x):
    b, s, d = x.shape
    idx = np.arange(M)
    starts = (idx * s) // M
    ends = -((-(idx + 1) * s) // M)
    csum = jnp.concatenate(
        [jnp.zeros((b, 1, d), dtype=x.dtype), jnp.cumsum(x, axis=1)], axis=1)
    counts = jnp.asarray(ends - starts, dtype=x.dtype)
    return (csum[:, ends, :] - csum[:, starts, :]) / counts[None, :, None]


def _kv_body(mem_ref, wk_ref, wv_ref, wp_ref, k_ref, vp_ref):
    mem = mem_ref[0]
    k_ref[...] = lax.dot_general(mem, wk_ref[...], (((1,), (1,)), ((), ())),
                                 preferred_element_type=jnp.float32)
    v = lax.dot_general(mem, wv_ref[...], (((1,), (1,)), ((), ())),
                        preferred_element_type=jnp.float32)
    vp_ref[...] = lax.dot_general(v, wp_ref[...], (((1,), (1,)), ((), ())),
                                  preferred_element_type=jnp.float32)


def _kv_call(mem3d, Wk, Wv, Wp):
    mblk = M // NSB
    return pl.pallas_call(
        _kv_body,
        grid=(NSB,),
        in_specs=[
            pl.BlockSpec((1, mblk, D), lambda i: (0, i, 0)),
            pl.BlockSpec((D, D), lambda i: (0, 0)),
            pl.BlockSpec((D, D), lambda i: (0, 0)),
            pl.BlockSpec((D, D), lambda i: (0, 0)),
        ],
        out_specs=[
            pl.BlockSpec((mblk, D), lambda i: (i, 0)),
            pl.BlockSpec((mblk, D), lambda i: (i, 0)),
        ],
        out_shape=[
            jax.ShapeDtypeStruct((M, D), jnp.float32),
            jax.ShapeDtypeStruct((M, D), jnp.float32),
        ],
    )(mem3d, Wk, Wv, Wp)


def _scores_body(x_ref, wq_ref, k_ref, st_ref):
    q = lax.dot_general(x_ref[0], wq_ref[...], (((1,), (1,)), ((), ())),
                        preferred_element_type=jnp.float32)
    st = lax.dot_general(k_ref[...], q, (((1,), (1,)), ((), ())),
                         preferred_element_type=jnp.float32)
    st_ref[...] = st * (1.0 / math.sqrt(D))


def _scores_call(x3d, Wq, kmat):
    return pl.pallas_call(
        _scores_body,
        grid=(NSB,),
        in_specs=[
            pl.BlockSpec((1, SBLK, D), lambda i: (0, i, 0)),
            pl.BlockSpec((D, D), lambda i: (0, 0)),
            pl.BlockSpec((M, D), lambda i: (0, 0)),
        ],
        out_specs=pl.BlockSpec((M, SBLK), lambda i: (0, i)),
        out_shape=jax.ShapeDtypeStruct((M, S), jnp.float32),
    )(x3d, Wq, kmat)


def _sc_topk_body(st_hbm, tau_hbm, mx_hbm, tile_f0, tile_f1, tile_k, cand,
                  hist, tau_buf, mx_buf, sem0, sem1):
    wid = lax.axis_index("s") * NC + lax.axis_index("c")
    lane = lax.iota(jnp.int32, L)
    ones = jnp.ones((L,), jnp.int32)
    zeros_i = jnp.zeros((L,), jnp.int32)
    r32 = jnp.full((L,), TOPK, jnp.int32)

    def zbody(b, _):
        hist[pl.ds(b * L, L)] = zeros_i
        return 0
    lax.fori_loop(0, 256, zbody, 0)

    def issue(g, buf, sem):
        col0 = (wid * GROUPS + g) * L
        pltpu.async_copy(st_hbm.at[:, pl.ds(col0, L)], buf, sem)

    def drain(buf, sem):
        pltpu.make_async_copy(st_hbm.at[:, pl.ds(0, L)], buf, sem).wait()

    def process(buf, g):
        def pa(mq, fmx):
            for u in range(4):
                m = mq * 4 + u
                v = buf[m, :]
                bu = plsc.bitcast(v, jnp.uint32)
                key = jnp.where(bu >= jnp.uint32(_UHI), ~bu,
                                bu | jnp.uint32(_UHI))
                tile_k[m, :] = plsc.bitcast(key, jnp.int32)
                fmx = jnp.maximum(fmx, v)
                byte = plsc.bitcast(key >> jnp.uint32(24), jnp.int32)
                plsc.addupdate_scatter(hist, [byte * L + lane], ones)
            return fmx
        fmax = lax.fori_loop(0, M // 4, pa,
                             jnp.full((L,), -jnp.inf, jnp.float32))

        def s1(iq, sc):
            cum, found, bsel, rrem = sc
            for u in range(4):
                b = 255 - (iq * 4 + u)
                cntv = hist[pl.ds(b * L, L)]
                hist[pl.ds(b * L, L)] = zeros_i
                newcum = cum + cntv
                nf = jnp.logical_and(found == 0, newcum >= r32)
                bsel = jnp.where(nf, b, bsel)
                rrem = jnp.where(nf, r32 - cum, rrem)
                found = jnp.where(nf, 1, found)
                cum = newcum
            return cum, found, bsel, rrem
        _, _, bsel, rank = lax.fori_loop(
            0, 64, s1, (zeros_i, zeros_i, zeros_i, zeros_i))

        def pc(mq, cnt):
            for u in range(4):
                m = mq * 4 + u
                ki = tile_k[m, :]
                byte = plsc.bitcast(
                    plsc.bitcast(ki, jnp.uint32) >> jnp.uint32(24), jnp.int32)
                c = byte == bsel
                plsc.store_scatter(cand, [cnt * L + lane], ki, mask=c)
                cnt = cnt + jnp.where(c, 1, 0)
            return cnt
        cnt = lax.fori_loop(0, M // 4, pc, zeros_i)
        cmax = lax.reduce_max(cnt, (0,))

        prefix = plsc.bitcast(bsel, jnp.uint32) << jnp.uint32(24)

        for p in range(6):
            shift = 20 - 4 * p
            himask = jnp.uint32((0xFFFFFFFF << (shift + 4)) & 0xFFFFFFFF)

            def ph(i, _, himask=himask, shift=shift, prefix=prefix, cnt=cnt):
                key = plsc.bitcast(cand[pl.ds(i * L, L)], jnp.uint32)
                valid = jnp.logical_and(cnt > i, (key & himask) == prefix)
                nib = plsc.bitcast(
                    (key >> jnp.uint32(shift)) & jnp.uint32(0xF), jnp.int32)
                plsc.addupdate_scatter(hist, [nib * L + lane], ones,
                                       mask=valid)
                return 0
            lax.fori_loop(0, cmax, ph, 0)

            def s2(iq, sc, rank=rank):
                cum, found, nsel, rrem = sc
                for u in range(4):
                    b = 15 - (iq * 4 + u)
                    cntv = hist[pl.ds(b * L, L)]
                    hist[pl.ds(b * L, L)] = zeros_i
                    newcum = cum + cntv
                    nf = jnp.logical_and(found == 0, newcum >= rank)
                    nsel = jnp.where(nf, b, nsel)
                    rrem = jnp.where(nf, rank - cum, rrem)
                    found = jnp.where(nf, 1, found)
                    cum = newcum
                return cum, found, nsel, rrem
            _, _, nsel, rank = lax.fori_loop(
                0, 4, s2, (zeros_i, zeros_i, zeros_i, zeros_i))
            prefix = prefix | (plsc.bitcast(nsel, jnp.uint32)
                               << jnp.uint32(shift))

        bits = jnp.where(prefix >= jnp.uint32(_UHI),
                         prefix ^ jnp.uint32(_UHI), ~prefix)
        tau = plsc.bitcast(bits, jnp.float32)
        tau_buf[pl.ds(g * L, L)] = tau
        mx_buf[pl.ds(g * L, L)] = fmax

    issue(0, tile_f0, sem0)

    def super_body(t, _):
        g0 = 2 * t
        drain(tile_f0, sem0)
        issue(g0 + 1, tile_f1, sem1)
        process(tile_f0, g0)
        drain(tile_f1, sem1)

        @pl.when(t < GROUPS // 2 - 1)
        def _prefetch():
            issue(g0 + 2, tile_f0, sem0)

        process(tile_f1, g0 + 1)
        return 0
    lax.fori_loop(0, GROUPS // 2, super_body, 0)

    pltpu.sync_copy(tau_buf, tau_hbm.at[pl.ds(wid * ROWS_PER_W, ROWS_PER_W)])
    pltpu.sync_copy(mx_buf, mx_hbm.at[pl.ds(wid * ROWS_PER_W, ROWS_PER_W)])


def _sc_topk_call(scoresT):
    mesh = plsc.VectorSubcoreMesh(core_axis_name="c", subcore_axis_name="s")
    fn = functools.partial(
        pl.kernel,
        out_type=(jax.ShapeDtypeStruct((S,), jnp.float32),
                  jax.ShapeDtypeStruct((S,), jnp.float32)),
        mesh=mesh,
        scratch_types=[
            pltpu.VMEM((M, L), jnp.float32),
            pltpu.VMEM((M, L), jnp.float32),
            pltpu.VMEM((M, L), jnp.int32),
            pltpu.VMEM((M * L,), jnp.int32),
            pltpu.VMEM((256 * L,), jnp.int32),
            pltpu.VMEM((ROWS_PER_W,), jnp.float32),
            pltpu.VMEM((ROWS_PER_W,), jnp.float32),
            pltpu.SemaphoreType.DMA,
            pltpu.SemaphoreType.DMA,
        ],
        compiler_params=pltpu.CompilerParams(use_tc_tiling_on_sc=False,
                                             needs_layout_passes=False),
    )(_sc_topk_body)
    return fn(scoresT)


def _combine_body(st_ref, tau_ref, mx_ref, vp_ref, out_ref):
    st = st_ref[...]
    tau = tau_ref[...]
    mx = mx_ref[...]
    w = jnp.where(st >= tau, jnp.exp(st - mx), 0.0)
    z = jnp.sum(w, axis=0, keepdims=True)
    wn = w / z
    out_ref[0] = lax.dot_general(wn, vp_ref[...], (((0,), (0,)), ((), ())),
                                 preferred_element_type=jnp.float32)


def _combine_call(scoresT, tau, mx, vp):
    return pl.pallas_call(
        _combine_body,
        grid=(NSB,),
        in_specs=[
            pl.BlockSpec((M, SBLK), lambda i: (0, i)),
            pl.BlockSpec((1, SBLK), lambda i: (0, i)),
            pl.BlockSpec((1, SBLK), lambda i: (0, i)),
            pl.BlockSpec((M, D), lambda i: (0, 0)),
        ],
        out_specs=pl.BlockSpec((1, SBLK, D), lambda i: (0, i, 0)),
        out_shape=jax.ShapeDtypeStruct((1, S, D), jnp.float32),
    )(scoresT, tau, mx, vp)


def kernel(x, Wq, Wk, Wv, Wp):
    b, s, d = x.shape
    assert (b, s, d) == (1, S, D)
    mem3d = _pool_slots(x)
    kmat, vp = _kv_call(mem3d, Wk, Wv, Wp)
    scoresT = _scores_call(x, Wq, kmat)
    tau, mx = _sc_topk_call(scoresT)
    return _combine_call(scoresT, tau.reshape(1, S), mx.reshape(1, S), vp)


# speedup vs baseline: 11.0811x; 1.8116x over previous
import functools
import math

import numpy as np

import jax
import jax.numpy as jnp
from jax import lax
from jax.experimental import pallas as pl
from jax.experimental.pallas import tpu as pltpu
from jax.experimental.pallas import tpu_sc as plsc

S = 4096
D = 1024
M = 1024
TOPK = 32

NC = 2
NS = 16
L = 16
NW = NC * NS
ROWS_PER_W = S // NW
GROUPS = ROWS_PER_W // L

SBLK = 512
NSB = S // SBLK

_UHI = 0x80000000


def _pool_slots(x):
    b, s, d = x.shape
    idx = np.arange(M)
    starts = (idx * s) // M
    ends = -((-(idx + 1) * s) // M)
    csum = jnp.concatenate(
        [jnp.zeros((b, 1, d), dtype=x.dtype), jnp.cumsum(x, axis=1)], axis=1)
    counts = jnp.asarray(ends - starts, dtype=x.dtype)
    return (csum[:, ends, :] - csum[:, starts, :]) / counts[None, :, None]


def _kv_body(mem_ref, wk_ref, wv_ref, wp_ref, k_ref, vp_ref):
    mem = mem_ref[0]
    k_ref[...] = lax.dot_general(mem, wk_ref[...], (((1,), (1,)), ((), ())),
                                 preferred_element_type=jnp.float32)
    v = lax.dot_general(mem, wv_ref[...], (((1,), (1,)), ((), ())),
                        preferred_element_type=jnp.float32)
    vp_ref[...] = lax.dot_general(v, wp_ref[...], (((1,), (1,)), ((), ())),
                                  preferred_element_type=jnp.float32)


def _kv_call(mem3d, Wk, Wv, Wp):
    mblk = M // NSB
    return pl.pallas_call(
        _kv_body,
        grid=(NSB,),
        in_specs=[
            pl.BlockSpec((1, mblk, D), lambda i: (0, i, 0)),
            pl.BlockSpec((D, D), lambda i: (0, 0)),
            pl.BlockSpec((D, D), lambda i: (0, 0)),
            pl.BlockSpec((D, D), lambda i: (0, 0)),
        ],
        out_specs=[
            pl.BlockSpec((mblk, D), lambda i: (i, 0)),
            pl.BlockSpec((mblk, D), lambda i: (i, 0)),
        ],
        out_shape=[
            jax.ShapeDtypeStruct((M, D), jnp.float32),
            jax.ShapeDtypeStruct((M, D), jnp.float32),
        ],
    )(mem3d, Wk, Wv, Wp)


def _scores_body(x_ref, wq_ref, k_ref, st_ref, mx_ref):
    q = lax.dot_general(x_ref[0], wq_ref[...], (((1,), (1,)), ((), ())),
                        preferred_element_type=jnp.float32)
    st = lax.dot_general(k_ref[...], q, (((1,), (1,)), ((), ())),
                         preferred_element_type=jnp.float32)
    st = st * (1.0 / math.sqrt(D))
    st_ref[...] = st
    mx_ref[...] = jnp.max(st, axis=0, keepdims=True)


def _scores_call(x3d, Wq, kmat):
    return pl.pallas_call(
        _scores_body,
        grid=(NSB,),
        in_specs=[
            pl.BlockSpec((1, SBLK, D), lambda i: (0, i, 0)),
            pl.BlockSpec((D, D), lambda i: (0, 0)),
            pl.BlockSpec((M, D), lambda i: (0, 0)),
        ],
        out_specs=[
            pl.BlockSpec((M, SBLK), lambda i: (0, i)),
            pl.BlockSpec((1, SBLK), lambda i: (0, i)),
        ],
        out_shape=[
            jax.ShapeDtypeStruct((M, S), jnp.float32),
            jax.ShapeDtypeStruct((1, S), jnp.float32),
        ],
    )(x3d, Wq, kmat)


def _mono_key(v):
    bi = plsc.bitcast(v, jnp.int32)
    flip = (bi >> 31) | jnp.int32(-0x80000000)
    return plsc.bitcast(bi ^ flip, jnp.uint32)


def _sc_topk_body(st_hbm, tau_hbm, tile_f0, tile_f1, cand, hist, tau_buf,
                  sem0, sem1):
    wid = lax.axis_index("s") * NC + lax.axis_index("c")
    lane = lax.iota(jnp.int32, L)
    ones = jnp.ones((L,), jnp.int32)
    zeros_i = jnp.zeros((L,), jnp.int32)
    r32 = jnp.full((L,), TOPK, jnp.int32)

    @plsc.parallel_loop(0, 256, 1, unroll=8)
    def _zero(b):
        hist[pl.ds(b * L, L)] = zeros_i

    def issue(g, buf, sem):
        col0 = (wid * GROUPS + g) * L
        pltpu.async_copy(st_hbm.at[:, pl.ds(col0, L)], buf, sem)

    def drain(buf, sem):
        pltpu.make_async_copy(st_hbm.at[:, pl.ds(0, L)], buf, sem).wait()

    def process(buf, g):
        @plsc.parallel_loop(0, M, 1, unroll=8)
        def _pa(m):
            key = _mono_key(buf[m, :])
            byte = plsc.bitcast(key >> jnp.uint32(24), jnp.int32)
            plsc.addupdate_scatter(hist, [byte * L + lane], ones)

        def s1(i, sc):
            cum, found, bsel, rrem = sc
            b = 255 - i
            cntv = hist[pl.ds(b * L, L)]
            hist[pl.ds(b * L, L)] = zeros_i
            newcum = cum + cntv
            nf = jnp.logical_and(found == 0, newcum >= r32)
            bsel = jnp.where(nf, b, bsel)
            rrem = jnp.where(nf, r32 - cum, rrem)
            found = jnp.where(nf, 1, found)
            return newcum, found, bsel, rrem
        _, _, bsel, rank = plsc.parallel_loop(
            0, 256, 1, unroll=8,
            carry=(zeros_i, zeros_i, zeros_i, zeros_i))(s1)

        def pc(m, cnt):
            key = _mono_key(buf[m, :])
            byte = plsc.bitcast(key >> jnp.uint32(24), jnp.int32)
            c = byte == bsel
            plsc.store_scatter(cand, [cnt * L + lane],
                               plsc.bitcast(key, jnp.int32), mask=c)
            return cnt + jnp.where(c, 1, 0)
        cnt = plsc.parallel_loop(0, M, 1, unroll=8, carry=zeros_i)(pc)
        cmax = lax.reduce_max(cnt, (0,))

        prefix = plsc.bitcast(bsel, jnp.uint32) << jnp.uint32(24)

        for p in range(6):
            shift = 20 - 4 * p
            himask = jnp.uint32((0xFFFFFFFF << (shift + 4)) & 0xFFFFFFFF)

            def ph(i, himask=himask, shift=shift, prefix=prefix, cnt=cnt):
                key = plsc.bitcast(cand[pl.ds(i * L, L)], jnp.uint32)
                valid = jnp.logical_and(cnt > i, (key & himask) == prefix)
                nib = plsc.bitcast(
                    (key >> jnp.uint32(shift)) & jnp.uint32(0xF), jnp.int32)
                plsc.addupdate_scatter(hist, [nib * L + lane], ones,
                                       mask=valid)
            plsc.parallel_loop(0, cmax, 1, unroll=4)(ph)

            def s2(i, sc, rank=rank):
                cum, found, nsel, rrem = sc
                b = 15 - i
                cntv = hist[pl.ds(b * L, L)]
                hist[pl.ds(b * L, L)] = zeros_i
                newcum = cum + cntv
                nf = jnp.logical_and(found == 0, newcum >= rank)
                nsel = jnp.where(nf, b, nsel)
                rrem = jnp.where(nf, rank - cum, rrem)
                found = jnp.where(nf, 1, found)
                return newcum, found, nsel, rrem
            _, _, nsel, rank = plsc.parallel_loop(
                0, 16, 1, unroll=4,
                carry=(zeros_i, zeros_i, zeros_i, zeros_i))(s2)
            prefix = prefix | (plsc.bitcast(nsel, jnp.uint32)
                               << jnp.uint32(shift))

        bits = jnp.where(prefix >= jnp.uint32(_UHI),
                         prefix ^ jnp.uint32(_UHI), ~prefix)
        tau = plsc.bitcast(bits, jnp.float32)
        tau_buf[pl.ds(g * L, L)] = tau

    issue(0, tile_f0, sem0)

    def super_body(t, _):
        g0 = 2 * t
        drain(tile_f0, sem0)
        issue(g0 + 1, tile_f1, sem1)
        process(tile_f0, g0)
        drain(tile_f1, sem1)

        @pl.when(t < GROUPS // 2 - 1)
        def _prefetch():
            issue(g0 + 2, tile_f0, sem0)

        process(tile_f1, g0 + 1)
        return 0
    lax.fori_loop(0, GROUPS // 2, super_body, 0)

    pltpu.sync_copy(tau_buf, tau_hbm.at[pl.ds(wid * ROWS_PER_W, ROWS_PER_W)])


def _sc_topk_call(scoresT):
    mesh = plsc.VectorSubcoreMesh(core_axis_name="c", subcore_axis_name="s")
    fn = functools.partial(
        pl.kernel,
        out_type=jax.ShapeDtypeStruct((S,), jnp.float32),
        mesh=mesh,
        scratch_types=[
            pltpu.VMEM((M, L), jnp.float32),
            pltpu.VMEM((M, L), jnp.float32),
            pltpu.VMEM((M * L,), jnp.int32),
            pltpu.VMEM((256 * L,), jnp.int32),
            pltpu.VMEM((ROWS_PER_W,), jnp.float32),
            pltpu.SemaphoreType.DMA,
            pltpu.SemaphoreType.DMA,
        ],
        compiler_params=pltpu.CompilerParams(use_tc_tiling_on_sc=False,
                                             needs_layout_passes=False),
    )(_sc_topk_body)
    return fn(scoresT)


def _combine_body(st_ref, tau_ref, mx_ref, vp_ref, out_ref):
    st = st_ref[...]
    tau = tau_ref[...]
    mx = mx_ref[...]
    w = jnp.where(st >= tau, jnp.exp(st - mx), 0.0)
    z = jnp.sum(w, axis=0, keepdims=True)
    wn = w / z
    out_ref[0] = lax.dot_general(wn, vp_ref[...], (((0,), (0,)), ((), ())),
                                 preferred_element_type=jnp.float32)


def _combine_call(scoresT, tau, mx, vp):
    return pl.pallas_call(
        _combine_body,
        grid=(NSB,),
        in_specs=[
            pl.BlockSpec((M, SBLK), lambda i: (0, i)),
            pl.BlockSpec((1, SBLK), lambda i: (0, i)),
            pl.BlockSpec((1, SBLK), lambda i: (0, i)),
            pl.BlockSpec((M, D), lambda i: (0, 0)),
        ],
        out_specs=pl.BlockSpec((1, SBLK, D), lambda i: (0, i, 0)),
        out_shape=jax.ShapeDtypeStruct((1, S, D), jnp.float32),
    )(scoresT, tau, mx, vp)


def kernel(x, Wq, Wk, Wv, Wp):
    b, s, d = x.shape
    assert (b, s, d) == (1, S, D)
    mem3d = _pool_slots(x)
    kmat, vp = _kv_call(mem3d, Wk, Wv, Wp)
    scoresT, mx = _scores_call(x, Wq, kmat)
    tau = _sc_topk_call(scoresT)
    return _combine_call(scoresT, tau.reshape(1, S), mx, vp)


# speedup vs baseline: 15.8959x; 1.4345x over previous
import functools
import math

import jax
import jax.numpy as jnp
from jax import lax
from jax.experimental import pallas as pl
from jax.experimental.pallas import tpu as pltpu
from jax.experimental.pallas import tpu_sc as plsc

S = 4096
D = 1024
M = 1024
TOPK = 32

NC = 2
NS = 16
L = 16
NW = NC * NS
ROWS_PER_W = S // NW
GROUPS = ROWS_PER_W // L

SBLK = 512
NSB = S // SBLK

_UHI = 0x80000000


def _kv_body(x_ref, wk_ref, wv_ref, wp_ref, k_ref, vp_ref, mem_s, c_s):
    i = pl.program_id(0)

    @pl.when(i == 0)
    def _init():
        c_s[...] = jnp.zeros((1, D), jnp.float32)

    C = c_s[...]
    for c in range(4):
        base = c * 128

        def jbody(j, carry, base=base, C=C):
            accW, prevE = carry
            r0 = base + 4 * j
            accW = accW + x_ref[0, pl.ds(r0, 1), :]
            accW = accW + x_ref[0, pl.ds(r0 + 1, 1), :]
            accW = accW + x_ref[0, pl.ds(r0 + 2, 1), :]
            accW = accW + x_ref[0, pl.ds(r0 + 3, 1), :]
            E = accW + C
            mem_s[pl.ds(c * 32 + j, 1), :] = (E - prevE) * 0.25
            return accW, E
        accW, _ = lax.fori_loop(
            0, 32, jbody, (jnp.zeros((1, D), jnp.float32), C))
        C = C + accW
    c_s[...] = C

    mem = mem_s[...]
    k_ref[...] = lax.dot_general(mem, wk_ref[...], (((1,), (1,)), ((), ())),
                                 preferred_element_type=jnp.float32)
    v = lax.dot_general(mem, wv_ref[...], (((1,), (1,)), ((), ())),
                        preferred_element_type=jnp.float32)
    vp_ref[...] = lax.dot_general(v, wp_ref[...], (((1,), (1,)), ((), ())),
                                  preferred_element_type=jnp.float32)


def _kv_call(x3d, Wk, Wv, Wp):
    mblk = M // NSB
    return pl.pallas_call(
        _kv_body,
        grid=(NSB,),
        in_specs=[
            pl.BlockSpec((1, 4 * mblk, D), lambda i: (0, i, 0)),
            pl.BlockSpec((D, D), lambda i: (0, 0)),
            pl.BlockSpec((D, D), lambda i: (0, 0)),
            pl.BlockSpec((D, D), lambda i: (0, 0)),
        ],
        out_specs=[
            pl.BlockSpec((mblk, D), lambda i: (i, 0)),
            pl.BlockSpec((mblk, D), lambda i: (i, 0)),
        ],
        out_shape=[
            jax.ShapeDtypeStruct((M, D), jnp.float32),
            jax.ShapeDtypeStruct((M, D), jnp.float32),
        ],
        scratch_shapes=[
            pltpu.VMEM((M // NSB, D), jnp.float32),
            pltpu.VMEM((1, D), jnp.float32),
        ],
    )(x3d, Wk, Wv, Wp)


def _scores_body(x_ref, wq_ref, k_ref, st_ref, mx_ref):
    q = lax.dot_general(x_ref[0], wq_ref[...], (((1,), (1,)), ((), ())),
                        preferred_element_type=jnp.float32)
    st = lax.dot_general(k_ref[...], q, (((1,), (1,)), ((), ())),
                         preferred_element_type=jnp.float32)
    st = st * (1.0 / math.sqrt(D))
    st_ref[...] = st
    mx_ref[...] = jnp.max(st, axis=0, keepdims=True)


def _scores_call(x3d, Wq, kmat):
    return pl.pallas_call(
        _scores_body,
        grid=(NSB,),
        in_specs=[
            pl.BlockSpec((1, SBLK, D), lambda i: (0, i, 0)),
            pl.BlockSpec((D, D), lambda i: (0, 0)),
            pl.BlockSpec((M, D), lambda i: (0, 0)),
        ],
        out_specs=[
            pl.BlockSpec((M, SBLK), lambda i: (0, i)),
            pl.BlockSpec((1, SBLK), lambda i: (0, i)),
        ],
        out_shape=[
            jax.ShapeDtypeStruct((M, S), jnp.float32),
            jax.ShapeDtypeStruct((1, S), jnp.float32),
        ],
    )(x3d, Wq, kmat)


def _mono_key(v):
    bi = plsc.bitcast(v, jnp.int32)
    flip = (bi >> 31) | jnp.int32(-0x80000000)
    return plsc.bitcast(bi ^ flip, jnp.uint32)


def _sc_topk_body(st_hbm, tau_hbm, tile_f0, tile_f1, cand, hist, tau_buf,
                  sem0, sem1):
    wid = lax.axis_index("s") * NC + lax.axis_index("c")
    lane = lax.iota(jnp.int32, L)
    ones = jnp.ones((L,), jnp.int32)
    zeros_i = jnp.zeros((L,), jnp.int32)
    r32 = jnp.full((L,), TOPK, jnp.int32)

    @plsc.parallel_loop(0, 256, 1, unroll=8)
    def _zero(b):
        hist[pl.ds(b * L, L)] = zeros_i

    def issue(g, buf, sem):
        col0 = (wid * GROUPS + g) * L
        pltpu.async_copy(st_hbm.at[:, pl.ds(col0, L)], buf, sem)

    def drain(buf, sem):
        pltpu.make_async_copy(st_hbm.at[:, pl.ds(0, L)], buf, sem).wait()

    def process(buf, g):
        @plsc.parallel_loop(0, M, 1, unroll=8)
        def _pa(m):
            key = _mono_key(buf[m, :])
            byte = plsc.bitcast(key >> jnp.uint32(24), jnp.int32)
            plsc.addupdate_scatter(hist, [byte * L + lane], ones)

        def s1(i, sc):
            cum, found, bsel, rrem = sc
            b = 255 - i
            cntv = hist[pl.ds(b * L, L)]
            hist[pl.ds(b * L, L)] = zeros_i
            newcum = cum + cntv
            nf = jnp.logical_and(found == 0, newcum >= r32)
            bsel = jnp.where(nf, b, bsel)
            rrem = jnp.where(nf, r32 - cum, rrem)
            found = jnp.where(nf, 1, found)
            return newcum, found, bsel, rrem
        _, _, bsel, rank = plsc.parallel_loop(
            0, 256, 1, unroll=8,
            carry=(zeros_i, zeros_i, zeros_i, zeros_i))(s1)

        def pc(m, cnt):
            key = _mono_key(buf[m, :])
            byte = plsc.bitcast(key >> jnp.uint32(24), jnp.int32)
            c = byte == bsel
            plsc.store_scatter(cand, [cnt * L + lane],
                               plsc.bitcast(key, jnp.int32), mask=c)
            return cnt + jnp.where(c, 1, 0)
        cnt = plsc.parallel_loop(0, M, 1, unroll=8, carry=zeros_i)(pc)
        cmax = lax.reduce_max(cnt, (0,))

        prefix = plsc.bitcast(bsel, jnp.uint32) << jnp.uint32(24)

        for p in range(6):
            shift = 20 - 4 * p
            himask = jnp.uint32((0xFFFFFFFF << (shift + 4)) & 0xFFFFFFFF)

            def ph(i, himask=himask, shift=shift, prefix=prefix, cnt=cnt):
                key = plsc.bitcast(cand[pl.ds(i * L, L)], jnp.uint32)
                valid = jnp.logical_and(cnt > i, (key & himask) == prefix)
                nib = plsc.bitcast(
                    (key >> jnp.uint32(shift)) & jnp.uint32(0xF), jnp.int32)
                plsc.addupdate_scatter(hist, [nib * L + lane], ones,
                                       mask=valid)
            plsc.parallel_loop(0, cmax, 1, unroll=4)(ph)

            def s2(i, sc, rank=rank):
                cum, found, nsel, rrem = sc
                b = 15 - i
                cntv = hist[pl.ds(b * L, L)]
                hist[pl.ds(b * L, L)] = zeros_i
                newcum = cum + cntv
                nf = jnp.logical_and(found == 0, newcum >= rank)
                nsel = jnp.where(nf, b, nsel)
                rrem = jnp.where(nf, rank - cum, rrem)
                found = jnp.where(nf, 1, found)
                return newcum, found, nsel, rrem
            _, _, nsel, rank = plsc.parallel_loop(
                0, 16, 1, unroll=4,
                carry=(zeros_i, zeros_i, zeros_i, zeros_i))(s2)
            prefix = prefix | (plsc.bitcast(nsel, jnp.uint32)
                               << jnp.uint32(shift))

        bits = jnp.where(prefix >= jnp.uint32(_UHI),
                         prefix ^ jnp.uint32(_UHI), ~prefix)
        tau = plsc.bitcast(bits, jnp.float32)
        tau_buf[pl.ds(g * L, L)] = tau

    issue(0, tile_f0, sem0)

    def super_body(t, _):
        g0 = 2 * t
        drain(tile_f0, sem0)
        issue(g0 + 1, tile_f1, sem1)
        process(tile_f0, g0)
        drain(tile_f1, sem1)

        @pl.when(t < GROUPS // 2 - 1)
        def _prefetch():
            issue(g0 + 2, tile_f0, sem0)

        process(tile_f1, g0 + 1)
        return 0
    lax.fori_loop(0, GROUPS // 2, super_body, 0)

    pltpu.sync_copy(tau_buf, tau_hbm.at[pl.ds(wid * ROWS_PER_W, ROWS_PER_W)])


def _sc_topk_call(scoresT):
    mesh = plsc.VectorSubcoreMesh(core_axis_name="c", subcore_axis_name="s")
    fn = functools.partial(
        pl.kernel,
        out_type=jax.ShapeDtypeStruct((S,), jnp.float32),
        mesh=mesh,
        scratch_types=[
            pltpu.VMEM((M, L), jnp.float32),
            pltpu.VMEM((M, L), jnp.float32),
            pltpu.VMEM((M * L,), jnp.int32),
            pltpu.VMEM((256 * L,), jnp.int32),
            pltpu.VMEM((ROWS_PER_W,), jnp.float32),
            pltpu.SemaphoreType.DMA,
            pltpu.SemaphoreType.DMA,
        ],
        compiler_params=pltpu.CompilerParams(use_tc_tiling_on_sc=False,
                                             needs_layout_passes=False),
    )(_sc_topk_body)
    return fn(scoresT)


def _combine_body(st_ref, tau_ref, mx_ref, vp_ref, out_ref):
    st = st_ref[...]
    tau = tau_ref[...]
    mx = mx_ref[...]
    w = jnp.where(st >= tau, jnp.exp(st - mx), 0.0)
    z = jnp.sum(w, axis=0, keepdims=True)
    wn = w / z
    out_ref[0] = lax.dot_general(wn, vp_ref[...], (((0,), (0,)), ((), ())),
                                 preferred_element_type=jnp.float32)


def _combine_call(scoresT, tau, mx, vp):
    return pl.pallas_call(
        _combine_body,
        grid=(NSB,),
        in_specs=[
            pl.BlockSpec((M, SBLK), lambda i: (0, i)),
            pl.BlockSpec((1, SBLK), lambda i: (0, i)),
            pl.BlockSpec((1, SBLK), lambda i: (0, i)),
            pl.BlockSpec((M, D), lambda i: (0, 0)),
        ],
        out_specs=pl.BlockSpec((1, SBLK, D), lambda i: (0, i, 0)),
        out_shape=jax.ShapeDtypeStruct((1, S, D), jnp.float32),
    )(scoresT, tau, mx, vp)


def kernel(x, Wq, Wk, Wv, Wp):
    b, s, d = x.shape
    assert (b, s, d) == (1, S, D)
    kmat, vp = _kv_call(x, Wk, Wv, Wp)
    scoresT, mx = _scores_call(x, Wq, kmat)
    tau = _sc_topk_call(scoresT)
    return _combine_call(scoresT, tau.reshape(1, S), mx, vp)


# speedup vs baseline: 18.7872x; 1.1819x over previous
import functools
import math

import jax
import jax.numpy as jnp
from jax import lax
from jax.experimental import pallas as pl
from jax.experimental.pallas import tpu as pltpu
from jax.experimental.pallas import tpu_sc as plsc

S = 4096
D = 1024
M = 1024
TOPK = 32

NC = 2
NS = 16
L = 16
NW = NC * NS
ROWS_PER_W = S // NW
GROUPS = ROWS_PER_W // L

SBLK = 1024
NSB = S // SBLK

_UHI = 0x80000000


def _kv_body(x_ref, wk_ref, wv_ref, wp_ref, k_ref, vp_ref, mem_s, c_s):
    i = pl.program_id(0)

    @pl.when(i == 0)
    def _init():
        c_s[...] = jnp.zeros((1, D), jnp.float32)

    C = c_s[...]
    for c in range(SBLK // 128):
        base = c * 128

        def jbody(j, carry, base=base, C=C):
            accW, prevE = carry
            r0 = base + 4 * j
            accW = accW + x_ref[0, pl.ds(r0, 1), :]
            accW = accW + x_ref[0, pl.ds(r0 + 1, 1), :]
            accW = accW + x_ref[0, pl.ds(r0 + 2, 1), :]
            accW = accW + x_ref[0, pl.ds(r0 + 3, 1), :]
            E = accW + C
            mem_s[pl.ds(c * 32 + j, 1), :] = (E - prevE) * 0.25
            return accW, E
        accW, _ = lax.fori_loop(
            0, 32, jbody, (jnp.zeros((1, D), jnp.float32), C))
        C = C + accW
    c_s[...] = C

    mem = mem_s[...]
    k_ref[...] = lax.dot_general(mem, wk_ref[...], (((1,), (1,)), ((), ())),
                                 preferred_element_type=jnp.float32)
    v = lax.dot_general(mem, wv_ref[...], (((1,), (1,)), ((), ())),
                        preferred_element_type=jnp.float32)
    vp_ref[...] = lax.dot_general(v, wp_ref[...], (((1,), (1,)), ((), ())),
                                  preferred_element_type=jnp.float32)


def _kv_call(x3d, Wk, Wv, Wp):
    mblk = M // NSB
    return pl.pallas_call(
        _kv_body,
        grid=(NSB,),
        in_specs=[
            pl.BlockSpec((1, 4 * mblk, D), lambda i: (0, i, 0)),
            pl.BlockSpec((D, D), lambda i: (0, 0)),
            pl.BlockSpec((D, D), lambda i: (0, 0)),
            pl.BlockSpec((D, D), lambda i: (0, 0)),
        ],
        out_specs=[
            pl.BlockSpec((mblk, D), lambda i: (i, 0)),
            pl.BlockSpec((mblk, D), lambda i: (i, 0)),
        ],
        out_shape=[
            jax.ShapeDtypeStruct((M, D), jnp.float32),
            jax.ShapeDtypeStruct((M, D), jnp.float32),
        ],
        scratch_shapes=[
            pltpu.VMEM((M // NSB, D), jnp.float32),
            pltpu.VMEM((1, D), jnp.float32),
        ],
    )(x3d, Wk, Wv, Wp)


def _scores_body(x_ref, wq_ref, k_ref, st_ref, mx_ref):
    q = lax.dot_general(x_ref[0], wq_ref[...], (((1,), (1,)), ((), ())),
                        preferred_element_type=jnp.float32)
    st = lax.dot_general(k_ref[...], q, (((1,), (1,)), ((), ())),
                         preferred_element_type=jnp.float32)
    st = st * (1.0 / math.sqrt(D))
    st_ref[...] = st.reshape(M, SBLK // 128, 128)
    mx_ref[...] = jnp.max(st, axis=0, keepdims=True)


def _scores_call(x3d, Wq, kmat):
    return pl.pallas_call(
        _scores_body,
        grid=(NSB,),
        in_specs=[
            pl.BlockSpec((1, SBLK, D), lambda i: (0, i, 0)),
            pl.BlockSpec((D, D), lambda i: (0, 0)),
            pl.BlockSpec((M, D), lambda i: (0, 0)),
        ],
        out_specs=[
            pl.BlockSpec((M, SBLK // 128, 128), lambda i: (0, i, 0)),
            pl.BlockSpec((1, SBLK), lambda i: (0, i)),
        ],
        out_shape=[
            jax.ShapeDtypeStruct((M, S // 128, 128), jnp.float32),
            jax.ShapeDtypeStruct((1, S), jnp.float32),
        ],
    )(x3d, Wq, kmat)


def _mono_key(v):
    bi = plsc.bitcast(v, jnp.int32)
    flip = (bi >> 31) | jnp.int32(-0x80000000)
    return plsc.bitcast(bi ^ flip, jnp.uint32)


def _sc_topk_body(st_hbm, tau_hbm, tile_f0, tile_f1, cand, hist, tau_buf,
                  sem0, sem1):
    wid = lax.axis_index("s") * NC + lax.axis_index("c")
    lane = lax.iota(jnp.int32, L)
    ones = jnp.ones((L,), jnp.int32)
    zeros_i = jnp.zeros((L,), jnp.int32)
    r32 = jnp.full((L,), TOPK, jnp.int32)

    @plsc.parallel_loop(0, 256, 1, unroll=8)
    def _zero(b):
        hist[pl.ds(b * L, L)] = zeros_i

    def issue(g, buf, sem):
        gg = wid * GROUPS + g
        pltpu.async_copy(
            st_hbm.at[:, gg // 8, pl.ds((gg % 8) * L, L)], buf, sem)

    def drain(buf, sem):
        pltpu.make_async_copy(st_hbm.at[:, 0, pl.ds(0, L)], buf, sem).wait()

    def process(buf, g):
        @plsc.parallel_loop(0, M, 1, unroll=8)
        def _pa(m):
            key = _mono_key(buf[m, :])
            byte = plsc.bitcast(key >> jnp.uint32(24), jnp.int32)
            plsc.addupdate_scatter(hist, [byte * L + lane], ones)

        def s1(i, sc):
            cum, found, bsel, rrem = sc
            b = 255 - i
            cntv = hist[pl.ds(b * L, L)]
            hist[pl.ds(b * L, L)] = zeros_i
            newcum = cum + cntv
            nf = jnp.logical_and(found == 0, newcum >= r32)
            bsel = jnp.where(nf, b, bsel)
            rrem = jnp.where(nf, r32 - cum, rrem)
            found = jnp.where(nf, 1, found)
            return newcum, found, bsel, rrem
        _, _, bsel, rank = plsc.parallel_loop(
            0, 256, 1, unroll=8,
            carry=(zeros_i, zeros_i, zeros_i, zeros_i))(s1)

        def pc(m, cnt):
            key = _mono_key(buf[m, :])
            byte = plsc.bitcast(key >> jnp.uint32(24), jnp.int32)
            c = byte == bsel
            plsc.store_scatter(cand, [cnt * L + lane],
                               plsc.bitcast(key, jnp.int32), mask=c)
            return cnt + jnp.where(c, 1, 0)
        cnt = plsc.parallel_loop(0, M, 1, unroll=8, carry=zeros_i)(pc)
        cmax = lax.reduce_max(cnt, (0,))

        prefix = plsc.bitcast(bsel, jnp.uint32) << jnp.uint32(24)

        for p in range(6):
            shift = 20 - 4 * p
            himask = jnp.uint32((0xFFFFFFFF << (shift + 4)) & 0xFFFFFFFF)

            def ph(i, himask=himask, shift=shift, prefix=prefix, cnt=cnt):
                key = plsc.bitcast(cand[pl.ds(i * L, L)], jnp.uint32)
                valid = jnp.logical_and(cnt > i, (key & himask) == prefix)
                nib = plsc.bitcast(
                    (key >> jnp.uint32(shift)) & jnp.uint32(0xF), jnp.int32)
                plsc.addupdate_scatter(hist, [nib * L + lane], ones,
                                       mask=valid)
            plsc.parallel_loop(0, cmax, 1, unroll=4)(ph)

            def s2(i, sc, rank=rank):
                cum, found, nsel, rrem = sc
                b = 15 - i
                cntv = hist[pl.ds(b * L, L)]
                hist[pl.ds(b * L, L)] = zeros_i
                newcum = cum + cntv
                nf = jnp.logical_and(found == 0, newcum >= rank)
                nsel = jnp.where(nf, b, nsel)
                rrem = jnp.where(nf, rank - cum, rrem)
                found = jnp.where(nf, 1, found)
                return newcum, found, nsel, rrem
            _, _, nsel, rank = plsc.parallel_loop(
                0, 16, 1, unroll=4,
                carry=(zeros_i, zeros_i, zeros_i, zeros_i))(s2)
            prefix = prefix | (plsc.bitcast(nsel, jnp.uint32)
                               << jnp.uint32(shift))

        bits = jnp.where(prefix >= jnp.uint32(_UHI),
                         prefix ^ jnp.uint32(_UHI), ~prefix)
        tau = plsc.bitcast(bits, jnp.float32)
        tau_buf[pl.ds(g * L, L)] = tau

    issue(0, tile_f0, sem0)

    def super_body(t, _):
        g0 = 2 * t
        drain(tile_f0, sem0)
        issue(g0 + 1, tile_f1, sem1)
        process(tile_f0, g0)
        drain(tile_f1, sem1)

        @pl.when(t < GROUPS // 2 - 1)
        def _prefetch():
            issue(g0 + 2, tile_f0, sem0)

        process(tile_f1, g0 + 1)
        return 0
    lax.fori_loop(0, GROUPS // 2, super_body, 0)

    pltpu.sync_copy(tau_buf, tau_hbm.at[pl.ds(wid * ROWS_PER_W, ROWS_PER_W)])


def _sc_topk_call(scoresT):
    mesh = plsc.VectorSubcoreMesh(core_axis_name="c", subcore_axis_name="s")
    fn = functools.partial(
        pl.kernel,
        out_type=jax.ShapeDtypeStruct((S,), jnp.float32),
        mesh=mesh,
        scratch_types=[
            pltpu.VMEM((M, L), jnp.float32),
            pltpu.VMEM((M, L), jnp.float32),
            pltpu.VMEM((M * L,), jnp.int32),
            pltpu.VMEM((256 * L,), jnp.int32),
            pltpu.VMEM((ROWS_PER_W,), jnp.float32),
            pltpu.SemaphoreType.DMA,
            pltpu.SemaphoreType.DMA,
        ],
        compiler_params=pltpu.CompilerParams(use_tc_tiling_on_sc=False,
                                             needs_layout_passes=False),
    )(_sc_topk_body)
    return fn(scoresT)


def _combine_body(st_ref, tau_ref, mx_ref, vp_ref, out_ref):
    st = st_ref[...].reshape(M, SBLK)
    tau = tau_ref[...]
    mx = mx_ref[...]
    w = jnp.where(st >= tau, jnp.exp(st - mx), 0.0)
    z = jnp.sum(w, axis=0, keepdims=True)
    wn = w / z
    out_ref[0] = lax.dot_general(wn, vp_ref[...], (((0,), (0,)), ((), ())),
                                 preferred_element_type=jnp.float32)


def _combine_call(scoresT, tau, mx, vp):
    return pl.pallas_call(
        _combine_body,
        grid=(NSB,),
        in_specs=[
            pl.BlockSpec((M, SBLK // 128, 128), lambda i: (0, i, 0)),
            pl.BlockSpec((1, SBLK), lambda i: (0, i)),
            pl.BlockSpec((1, SBLK), lambda i: (0, i)),
            pl.BlockSpec((M, D), lambda i: (0, 0)),
        ],
        out_specs=pl.BlockSpec((1, SBLK, D), lambda i: (0, i, 0)),
        out_shape=jax.ShapeDtypeStruct((1, S, D), jnp.float32),
    )(scoresT, tau, mx, vp)


def kernel(x, Wq, Wk, Wv, Wp):
    b, s, d = x.shape
    assert (b, s, d) == (1, S, D)
    kmat, vp = _kv_call(x, Wk, Wv, Wp)
    scoresT, mx = _scores_call(x, Wq, kmat)
    tau = _sc_topk_call(scoresT)
    return _combine_call(scoresT, tau.reshape(1, S), mx, vp)
